# baseline (speedup 1.0000x reference)
import functools
import math

import jax
import jax.numpy as jnp
from jax import lax
from jax.experimental import pallas as pl
from jax.experimental.pallas import tpu as pltpu
from jax.experimental.pallas import tpu_sc as plsc

EMB = 64
SCALE = math.sqrt(float(EMB))
NC = 2
NS = 16
NW = NC * NS
LANES = 16
BCHUNK = 4
GROUP = 4


def _xlane_sum(x):
    for d in (1, 2, 4, 8):
        perm = lax.iota(jnp.int32, LANES) ^ d
        x = x + x.at[perm].get(mode="promise_in_bounds")
    return x


def _unpack_pair(w):
    lo = lax.bitcast_convert_type(lax.shift_left(w, 16), jnp.float32)
    hi = lax.bitcast_convert_type(w & jnp.int32(-65536), jnp.float32)
    return lo, hi


def _row_update(rows_v, out_v, tok_v, r):
    col = (tok_v[pl.ds(r, LANES)][0] & (GROUP - 1)) * (2 * LANES)
    wa = rows_v[r, pl.ds(col, LANES)]
    wb = rows_v[r, pl.ds(col + LANES, LANES)]
    v0, v1 = _unpack_pair(wa)
    v2, v3 = _unpack_pair(wb)
    ss = v0 * v0 + v1 * v1 + v2 * v2 + v3 * v3
    tv = _xlane_sum(ss)
    m = jnp.maximum(tv, 1.0)
    i = lax.bitcast_convert_type(m, jnp.int32)
    i = jnp.int32(0x5F3759DF) - lax.shift_right_arithmetic(i, 1)
    y = lax.bitcast_convert_type(i, jnp.float32)
    h = 0.5 * m
    y = y * (1.5 - h * y * y)
    y = y * (1.5 - h * y * y)
    f = y * SCALE
    out_v[r, pl.ds(0, LANES)] = v0 * f
    out_v[r, pl.ds(LANES, LANES)] = v1 * f
    out_v[r, pl.ds(2 * LANES, LANES)] = v2 * f
    out_v[r, pl.ds(3 * LANES, LANES)] = v3 * f


@functools.partial(jax.jit, static_argnames=("b", "l"))
def _emb_lookup(tokens_flat, table_pk, *, b, l):
    n = b * l
    per_w = n // NW
    b_per_w = b // NW
    nchunk = b_per_w // BCHUNK
    chunk = BCHUNK * l

    mesh = plsc.VectorSubcoreMesh(core_axis_name="c", subcore_axis_name="s")

    @functools.partial(
        pl.kernel,
        mesh=mesh,
        out_type=jax.ShapeDtypeStruct((b, l, EMB), jnp.float32),
        scratch_types=[
            pltpu.VMEM((chunk + LANES,), jnp.int32),
            pltpu.VMEM((chunk + LANES,), jnp.int32),
            pltpu.VMEM((chunk,), jnp.int32),
            pltpu.VMEM((chunk, 2 * EMB), jnp.int32),
            pltpu.VMEM((chunk, 2 * EMB), jnp.int32),
            pltpu.VMEM((chunk, EMB), jnp.float32),
            pltpu.SemaphoreType.DMA,
        ],
    )
    def body(
        tok_hbm, table_hbm, out_hbm,
        tok_v0, tok_v1, idx_v, rows_v0, rows_v1, out_v, sem,
    ):
        wid = lax.axis_index("s") * NC + lax.axis_index("c")
        base = wid * per_w
        bbase = wid * b_per_w
        nidx = chunk // LANES + (1 if chunk % LANES else 0)
        toks = (tok_v0, tok_v1)
        rows = (rows_v0, rows_v1)

        def fetch(g, p):
            tok_v = toks[p]
            off = base + g * chunk
            pltpu.sync_copy(
                tok_hbm.at[pl.ds(off, chunk)], tok_v.at[pl.ds(0, chunk)]
            )

            @plsc.parallel_loop(0, nidx, unroll=4)
            def make_idx(i):
                s = jnp.minimum(i * LANES, chunk - LANES)
                t = tok_v[pl.ds(s, LANES)]
                idx_v[pl.ds(s, LANES)] = lax.shift_right_logical(t, 2)

            return pltpu.make_async_copy(table_hbm.at[idx_v], rows[p], sem)

        def consume(g, p):
            b0 = bbase + g * BCHUNK

            @plsc.parallel_loop(0, chunk, unroll=8)
            def do_row(r):
                _row_update(rows[p], out_v, toks[p], r)

            for bb in range(BCHUNK):
                pltpu.sync_copy(
                    out_v.at[pl.ds(bb * l, l)], out_hbm.at[b0 + bb]
                )

        fetch(0, 0).start()

        def do_pair(h, carry):
            for p in (0, 1):
                g = 2 * h + p
                pltpu.make_async_copy(table_hbm.at[idx_v], rows[p], sem).wait()
                gn = jnp.minimum(g + 1, nchunk - 1)
                fetch(gn, 1 - p).start()
                consume(g, p)
            return carry

        lax.fori_loop(0, nchunk // 2, do_pair, 0)
        pltpu.make_async_copy(table_hbm.at[idx_v], rows[0], sem).wait()

    return body(tokens_flat, table_pk)


def kernel(tokens, table):
    b, l = tokens.shape
    flat = tokens.reshape(-1).astype(jnp.int32)
    v = table.shape[0]
    u = lax.bitcast_convert_type(
        table.astype(jnp.bfloat16).reshape(v, 2, 2, LANES), jnp.uint16
    )
    w = u[:, :, 0, :].astype(jnp.uint32) | (
        u[:, :, 1, :].astype(jnp.uint32) << 16
    )
    table_pk = lax.bitcast_convert_type(w, jnp.int32).reshape(
        v // GROUP, GROUP * EMB // 2
    )
    return _emb_lookup(flat, table_pk, b=b, l=l)
